# baseline (speedup 1.0000x reference)
import functools

import jax
import jax.numpy as jnp
from jax import lax
from jax.experimental import pallas as pl
from jax.experimental.pallas import tpu as pltpu
from jax.experimental.pallas import tpu_sc as plsc

NC, NS, L = 2, 16, 16
NW = NC * NS
D = 128
B, TPH = 1024, 200
N = B * TPH
PW = N // NW
W = 128
NWIN = PW // W
NBUF = 5
NT, NB = 8, 6
NCB = NT * NB
VP = 1024
RPT = VP // NS
CCH = D // L


def _make_kernel():
    mesh = plsc.VectorSubcoreMesh(core_axis_name="c", subcore_axis_name="s")

    scratch = (
        [pltpu.VMEM((NWIN, W), jnp.int32)] * 3
        + [pltpu.VMEM((NT, D), jnp.float32),
           pltpu.VMEM((NB, D), jnp.float32),
           pltpu.VMEM((NCB, D), jnp.float32)]
        + [pltpu.VMEM((W, D), jnp.float32)] * NBUF
        + [pltpu.VMEM_SHARED((VP, D), jnp.float32)]
        + [pltpu.SemaphoreType.DMA] * (2 * NBUF)
    )

    @functools.partial(
        pl.kernel,
        out_type=jax.ShapeDtypeStruct((N, D), jnp.float32),
        mesh=mesh,
        compiler_params=pltpu.CompilerParams(needs_layout_passes=False),
        scratch_types=scratch,
    )
    def k(ph_ids_hbm, tone_ids_hbm, bnd_ids_hbm,
          ph_tab_hbm, tone_tab_hbm, bnd_tab_hbm,
          out_hbm,
          ids_v, tid_v, bid_v, tone_tab_v, bnd_tab_v, combo_v, *bufs_and_sems):
        rows = bufs_and_sems[:NBUF]
        tab_sp = bufs_and_sems[NBUF]
        gsems = bufs_and_sems[NBUF + 1:NBUF + 1 + NBUF]
        osems = bufs_and_sems[NBUF + 1 + NBUF:]

        wid = lax.axis_index("s") * NC + lax.axis_index("c")
        base = wid * PW

        pltpu.sync_copy(ph_ids_hbm.at[wid], ids_v)
        pltpu.sync_copy(tone_ids_hbm.at[wid], tid_v)
        pltpu.sync_copy(bnd_ids_hbm.at[wid], bid_v)
        pltpu.sync_copy(tone_tab_hbm, tone_tab_v)
        pltpu.sync_copy(bnd_tab_hbm, bnd_tab_v)

        sid = lax.axis_index("s")
        pltpu.sync_copy(ph_tab_hbm.at[pl.ds(sid * RPT, RPT)],
                        tab_sp.at[pl.ds(sid * RPT, RPT)])
        plsc.subcore_barrier()

        def build_combo(i, carry):
            t = i // NB
            b = i - t * NB
            for c in range(CCH):
                v = (tone_tab_v[t, pl.ds(c * L, L)]
                     + bnd_tab_v[b, pl.ds(c * L, L)])
                combo_v[i, pl.ds(c * L, L)] = v
            return carry
        lax.fori_loop(0, NCB, build_combo, 0)

        def build_cid(i, carry):
            r = i // CCH
            kk = i - r * CCH
            t = tid_v[r, pl.ds(kk * L, L)]
            b = bid_v[r, pl.ds(kk * L, L)]
            tid_v[r, pl.ds(kk * L, L)] = t * NB + b
            return carry
        lax.fori_loop(0, NWIN * CCH, build_cid, 0)

        iota = lax.iota(jnp.int32, L)
        cols = [iota + (c * L) for c in range(CCH)]

        def g_copy(w, p):
            return pltpu.make_async_copy(
                tab_sp.at[ids_v.at[w]], rows[p], gsems[p])

        def o_copy(w, p):
            return pltpu.make_async_copy(
                rows[p], out_hbm.at[pl.ds(base + w * W, W)], osems[p])

        def compute(w, p):
            def chunk(ck, carry2):
                cvec = tid_v[w, pl.ds(ck * L, L)]
                for j in range(L):
                    cb = jnp.take_along_axis(
                        cvec, jnp.full((L,), j, jnp.int32), axis=0,
                        mode="promise_in_bounds")
                    pos = ck * L + j
                    for c in range(CCH):
                        val = plsc.load_gather(combo_v, [cb, cols[c]])
                        plsc.addupdate(rows[p].at[pos, pl.ds(c * L, L)], val)
                return carry2
            lax.fori_loop(0, CCH, chunk, 0)

        def step(w, par, do_owait, do_gstart):
            g_copy(w, par).wait()
            if do_owait:
                o_copy(w - 3, (par + 2) % NBUF).wait()
            if do_gstart:
                g_copy(w + 2, (par + 2) % NBUF).start()
            compute(w, par)
            o_copy(w, par).start()

        g_copy(0, 0).start()
        g_copy(1, 1).start()

        for par in range(NBUF):
            step(par, par, par >= 3, True)

        def round_body(r, carry):
            w0 = r * NBUF
            for par in range(NBUF):
                step(w0 + par, par, True, True)
            return carry
        lax.fori_loop(1, NWIN // NBUF - 1, round_body, 0)

        w0 = NWIN - NBUF
        for par in range(NBUF):
            w = w0 + par
            step(w, par, True, w + 2 < NWIN)

        for w in (NWIN - 3, NWIN - 2, NWIN - 1):
            o_copy(w, w % NBUF).wait()

    return k


_kernel_fn = _make_kernel()


@jax.jit
def _run(ph_ids, tone_ids, boundary_ids, ph_table, tone_table, boundary_table):
    ph = ph_ids.reshape(NW, NWIN, W).astype(jnp.int32)
    ph_table = jnp.concatenate(
        [ph_table, jnp.zeros((VP - ph_table.shape[0], D), ph_table.dtype)])
    tn = tone_ids.reshape(NW, NWIN, W).astype(jnp.int32)
    bd = boundary_ids.reshape(NW, NWIN, W).astype(jnp.int32)
    out = _kernel_fn(ph, tn, bd, ph_table, tone_table, boundary_table)
    return out.reshape(B, TPH, D)


def kernel(ph_ids, tone_ids, boundary_ids, ph_table, tone_table, boundary_table):
    return _run(ph_ids, tone_ids, boundary_ids, ph_table, tone_table,
                boundary_table)


# speedup vs baseline: 1.0275x; 1.0275x over previous
import functools

import jax
import jax.numpy as jnp
from jax import lax
from jax.experimental import pallas as pl
from jax.experimental.pallas import tpu as pltpu
from jax.experimental.pallas import tpu_sc as plsc

NC, NS, L = 2, 16, 16
NW = NC * NS
D = 128
B, TPH = 1024, 200
N = B * TPH
PW = N // NW
WI = 128
GPW = 1
W = WI * GPW
NWIN = PW // W
NROW = PW // WI
NBUF = 3
NT, NB = 8, 6
NCB = NT * NB
CCH = D // L
VP = 1024
RPT = VP // NS


def _make_kernel():
    mesh = plsc.VectorSubcoreMesh(core_axis_name="c", subcore_axis_name="s")

    scratch = (
        [pltpu.VMEM((NROW, WI), jnp.int32)] * 3
        + [pltpu.VMEM((NT, D), jnp.float32),
           pltpu.VMEM((NB, D), jnp.float32),
           pltpu.VMEM((NCB, D), jnp.float32)]
        + [pltpu.VMEM((W, D), jnp.float32)] * NBUF
        + [pltpu.VMEM_SHARED((VP, D), jnp.float32)]
        + [pltpu.SemaphoreType.DMA] * (2 * NBUF)
    )

    @functools.partial(
        pl.kernel,
        out_type=jax.ShapeDtypeStruct((N, D), jnp.float32),
        mesh=mesh,
        compiler_params=pltpu.CompilerParams(needs_layout_passes=False),
        scratch_types=scratch,
    )
    def k(ph_ids_hbm, tone_ids_hbm, bnd_ids_hbm,
          ph_tab_hbm, tone_tab_hbm, bnd_tab_hbm,
          out_hbm,
          ids_v, tid_v, bid_v, tone_tab_v, bnd_tab_v, combo_v, *rest):
        rows = rest[:NBUF]
        tab_sp = rest[NBUF]
        gsems = rest[NBUF + 1:2 * NBUF + 1]
        osems = rest[2 * NBUF + 1:]

        wid = lax.axis_index("s") * NC + lax.axis_index("c")
        base = wid * PW

        pltpu.sync_copy(ph_ids_hbm.at[wid], ids_v)
        pltpu.sync_copy(tone_ids_hbm.at[wid], tid_v)
        pltpu.sync_copy(bnd_ids_hbm.at[wid], bid_v)
        pltpu.sync_copy(tone_tab_hbm, tone_tab_v)
        pltpu.sync_copy(bnd_tab_hbm, bnd_tab_v)

        sid = lax.axis_index("s")
        pltpu.sync_copy(ph_tab_hbm.at[pl.ds(sid * RPT, RPT)],
                        tab_sp.at[pl.ds(sid * RPT, RPT)])
        plsc.subcore_barrier()

        def build_combo(i, carry):
            t = i // NB
            b = i - t * NB
            for c in range(CCH):
                v = (tone_tab_v[t, pl.ds(c * L, L)]
                     + bnd_tab_v[b, pl.ds(c * L, L)])
                combo_v[i, pl.ds(c * L, L)] = v
            return carry
        lax.fori_loop(0, NCB, build_combo, 0)

        def build_cid(i, carry):
            r = i // (WI // L)
            kk = i - r * (WI // L)
            t = tid_v[r, pl.ds(kk * L, L)]
            b = bid_v[r, pl.ds(kk * L, L)]
            tid_v[r, pl.ds(kk * L, L)] = t * NB + b
            return carry
        lax.fori_loop(0, NROW * (WI // L), build_cid, 0)

        iota = lax.iota(jnp.int32, L)
        cols = [iota + (c * L) for c in range(CCH)]

        def g_start(w, p):
            for h in range(GPW):
                pltpu.async_copy(tab_sp.at[ids_v.at[GPW * w + h]],
                                 rows[p].at[pl.ds(h * WI, WI)], gsems[p])

        def g_wait(w, p):
            for h in range(GPW):
                pltpu.make_async_copy(
                    tab_sp.at[ids_v.at[GPW * w + h]],
                    rows[p].at[pl.ds(h * WI, WI)], gsems[p]).wait()

        def o_copy(w, p):
            return pltpu.make_async_copy(
                rows[p], out_hbm.at[pl.ds(base + w * W, W)], osems[p])

        def compute(w, p):
            def chunk(ck, carry2):
                rr = GPW * w + ck // (WI // L)
                kk = ck - (ck // (WI // L)) * (WI // L)
                cvec = tid_v[rr, pl.ds(kk * L, L)]
                for j in range(L):
                    cb = jnp.take_along_axis(
                        cvec, jnp.full((L,), j, jnp.int32), axis=0,
                        mode="promise_in_bounds")
                    pos = ck * L + j
                    for c in range(CCH):
                        val = plsc.load_gather(combo_v, [cb, cols[c]])
                        plsc.addupdate(rows[p].at[pos, pl.ds(c * L, L)], val)
                return carry2
            lax.fori_loop(0, W // L, chunk, 0)

        def step(w, par, do_owait, do_gstart):
            g_wait(w, par)
            if do_owait:
                o_copy(w - 2, (par + 1) % NBUF).wait()
            if do_gstart:
                g_start(w + 1, (par + 1) % NBUF)
            compute(w, par)
            o_copy(w, par).start()

        g_start(0, 0)

        for par in range(NBUF):
            step(par, par, par >= 2, True)

        def round_body(r, carry):
            w0 = r * NBUF
            for par in range(NBUF):
                step(w0 + par, par, True, True)
            return carry
        lax.fori_loop(1, (NWIN - 1) // NBUF, round_body, 0)

        for w in range(((NWIN - 1) // NBUF) * NBUF, NWIN):
            step(w, w % NBUF, True, w + 1 < NWIN)

        for w in (NWIN - 2, NWIN - 1):
            o_copy(w, w % NBUF).wait()

    return k


_kernel_fn = _make_kernel()


@jax.jit
def _run(ph_ids, tone_ids, boundary_ids, ph_table, tone_table, boundary_table):
    ph = ph_ids.reshape(NW, NROW, WI).astype(jnp.int32)
    tn = tone_ids.reshape(NW, NROW, WI).astype(jnp.int32)
    bd = boundary_ids.reshape(NW, NROW, WI).astype(jnp.int32)
    ph_table = jnp.concatenate(
        [ph_table, jnp.zeros((VP - ph_table.shape[0], D), ph_table.dtype)])
    out = _kernel_fn(ph, tn, bd, ph_table, tone_table, boundary_table)
    return out.reshape(B, TPH, D)


def kernel(ph_ids, tone_ids, boundary_ids, ph_table, tone_table, boundary_table):
    return _run(ph_ids, tone_ids, boundary_ids, ph_table, tone_table,
                boundary_table)


# speedup vs baseline: 1.3294x; 1.2939x over previous
import functools

import jax
import jax.numpy as jnp
from jax import lax
from jax.experimental import pallas as pl
from jax.experimental.pallas import tpu as pltpu
from jax.experimental.pallas import tpu_sc as plsc

NC, NS, L = 2, 16, 16
NW = NC * NS
D = 128
B, TPH = 1024, 200
N = B * TPH
PW = N // NW
WI = 128
W = WI
NWIN = PW // W
NROW = PW // WI
NBUF = 3
NT, NB = 8, 6
CCH = D // L
VP = 1024
RPT = VP // NS
MROWS = NT * VP


def _make_kernel():
    mesh = plsc.VectorSubcoreMesh(core_axis_name="c", subcore_axis_name="s")

    scratch = (
        [pltpu.VMEM((NROW, WI), jnp.int32)] * 2
        + [pltpu.VMEM((NB, D), jnp.float32)]
        + [pltpu.VMEM((W, D), jnp.float32)] * NBUF
        + [pltpu.VMEM_SHARED((MROWS, D), jnp.float32)]
        + [pltpu.SemaphoreType.DMA] * (2 * NBUF)
    )

    @functools.partial(
        pl.kernel,
        out_type=jax.ShapeDtypeStruct((N, D), jnp.float32),
        mesh=mesh,
        compiler_params=pltpu.CompilerParams(needs_layout_passes=False),
        scratch_types=scratch,
    )
    def k(ph_ids_hbm, tone_ids_hbm, bnd_ids_hbm,
          ph_tab_hbm, tone_tab_hbm, bnd_tab_hbm,
          out_hbm,
          ids_v, bid_v, bnd_tab_v, *rest):
        rows = rest[:NBUF]
        tab_sp = rest[NBUF]
        gsems = rest[NBUF + 1:2 * NBUF + 1]
        osems = rest[2 * NBUF + 1:]

        wid = lax.axis_index("s") * NC + lax.axis_index("c")
        base = wid * PW

        pltpu.sync_copy(ph_ids_hbm.at[wid], ids_v)
        pltpu.sync_copy(tone_ids_hbm.at[wid], bid_v)
        def build_fid(i, carry):
            r = i // (WI // L)
            kk = i - r * (WI // L)
            p = ids_v[r, pl.ds(kk * L, L)]
            t = bid_v[r, pl.ds(kk * L, L)]
            ids_v[r, pl.ds(kk * L, L)] = t * VP + p
            return carry
        lax.fori_loop(0, NROW * (WI // L), build_fid, 0)
        pltpu.sync_copy(bnd_ids_hbm.at[wid], bid_v)
        pltpu.sync_copy(bnd_tab_hbm, bnd_tab_v)

        sid = lax.axis_index("s")
        pltpu.sync_copy(ph_tab_hbm.at[pl.ds(sid * RPT, RPT)],
                        rows[0].at[pl.ds(0, RPT)])
        pltpu.sync_copy(tone_tab_hbm, rows[1].at[pl.ds(0, NT)])
        for t in range(NT):
            dt = []
            for c in range(CCH):
                v = rows[1][t, pl.ds(c * L, L)]
                if t > 0:
                    v = v - rows[1][t - 1, pl.ds(c * L, L)]
                dt.append(v)

            def add_dt(r, carry):
                for c in range(CCH):
                    plsc.addupdate(rows[0].at[r, pl.ds(c * L, L)], dt[c])
                return carry
            lax.fori_loop(0, RPT, add_dt, 0)
            pltpu.sync_copy(rows[0].at[pl.ds(0, RPT)],
                            tab_sp.at[pl.ds(t * VP + sid * RPT, RPT)])
        plsc.subcore_barrier()

        def g_start(w, p):
            pltpu.async_copy(tab_sp.at[ids_v.at[w]], rows[p], gsems[p])

        def g_wait(w, p):
            pltpu.make_async_copy(tab_sp.at[ids_v.at[w]], rows[p],
                                  gsems[p]).wait()

        def o_copy(w, p):
            return pltpu.make_async_copy(
                rows[p], out_hbm.at[pl.ds(base + w * W, W)], osems[p])

        def compute(w, p):
            def chunk(i, carry2):
                ck = i // CCH
                c = i - ck * CCH
                coff = c * L
                bvec = bid_v[w, pl.ds(ck * L, L)]
                bv = [bnd_tab_v[kk, pl.ds(coff, L)] for kk in range(NB)]
                for j in range(L):
                    bb = jnp.take_along_axis(
                        bvec, jnp.full((L,), j, jnp.int32), axis=0,
                        mode="promise_in_bounds")
                    val = bv[NB - 1]
                    for kk in range(NB - 2, -1, -1):
                        val = jnp.where(bb == kk, bv[kk], val)
                    plsc.addupdate(rows[p].at[ck * L + j, pl.ds(coff, L)],
                                   val)
                return carry2
            lax.fori_loop(0, (W // L) * CCH, chunk, 0)

        def step(w, par, do_owait, do_gstart):
            g_wait(w, par)
            if do_owait:
                o_copy(w - 2, (par + 1) % NBUF).wait()
            if do_gstart:
                g_start(w + 1, (par + 1) % NBUF)
            compute(w, par)
            o_copy(w, par).start()

        g_start(0, 0)

        for par in range(NBUF):
            step(par, par, par >= 2, True)

        def round_body(r, carry):
            w0 = r * NBUF
            for par in range(NBUF):
                step(w0 + par, par, True, True)
            return carry
        lax.fori_loop(1, (NWIN - 1) // NBUF, round_body, 0)

        for w in range(((NWIN - 1) // NBUF) * NBUF, NWIN):
            step(w, w % NBUF, True, w + 1 < NWIN)

        for w in (NWIN - 2, NWIN - 1):
            o_copy(w, w % NBUF).wait()

    return k


_kernel_fn = _make_kernel()


@jax.jit
def _run(ph_ids, tone_ids, boundary_ids, ph_table, tone_table, boundary_table):
    ph = ph_ids.reshape(NW, NROW, WI).astype(jnp.int32)
    tn = tone_ids.reshape(NW, NROW, WI).astype(jnp.int32)
    bd = boundary_ids.reshape(NW, NROW, WI).astype(jnp.int32)
    ph_table = jnp.concatenate(
        [ph_table, jnp.zeros((VP - ph_table.shape[0], D), ph_table.dtype)])
    out = _kernel_fn(ph, tn, bd, ph_table, tone_table, boundary_table)
    return out.reshape(B, TPH, D)


def kernel(ph_ids, tone_ids, boundary_ids, ph_table, tone_table, boundary_table):
    return _run(ph_ids, tone_ids, boundary_ids, ph_table, tone_table,
                boundary_table)
